# Initial kernel scaffold; baseline (speedup 1.0000x reference)
import jax
import jax.numpy as jnp
from jax.experimental import pallas as pl


def kernel(points, cost, default_cost):
    raise NotImplementedError("write your pallas kernel here")


# speedup vs baseline: 2.2513x; 2.2513x over previous
import jax
import jax.numpy as jnp
from jax import lax
from jax.experimental import pallas as pl
from jax.experimental.pallas import tpu as pltpu
from jax.experimental.pallas import tpu_sc as plsc

H = 1024
W = 1024
N = 2000000
HW = H * W
HALF = HW // 2
NC = 2
NS = 16
L = 16

Q = N // NS
CHUNK = 1024
GROUPS = CHUNK // L
NBATCH = CHUNK // 128
NCH = -(-Q // CHUNK)
LAST_S = Q - CHUNK

RPT = HALF // NS
FCH = 2048
NFCH = RPT // FCH
ZLEN = 4096


def _body(points_hbm, cost_hbm, dflt_hbm, map_hbm, mask_hbm,
          acc_s, acc_c, pts_v, cost_v, vals_s, vals_c, idx_v,
          fin_s, fin_c, omap_v, omask_v, dflt_v, zsrc, sem):
    cid = lax.axis_index("c")
    sid = lax.axis_index("s")
    iota = lax.iota(jnp.int32, L)
    fzero = jnp.zeros((L,), jnp.float32)

    def z_body(g, carry):
        plsc.store_scatter(zsrc, [g * L + iota], fzero)
        return carry
    lax.fori_loop(0, ZLEN // L, z_body, 0)
    base0 = sid * RPT
    for k in range(RPT // ZLEN):
        pltpu.sync_copy(zsrc, acc_s.at[pl.ds(base0 + k * ZLEN, ZLEN)])
        pltpu.sync_copy(zsrc, acc_c.at[pl.ds(base0 + k * ZLEN, ZLEN)])

    pltpu.sync_copy(dflt_hbm, dflt_v)
    dflt = dflt_v[...]

    plsc.subcore_barrier()

    pbase = sid * Q

    def chunk_body(c, carry):
        s = jnp.minimum(c * CHUNK, LAST_S)
        thr = c * CHUNK - s
        pltpu.sync_copy(points_hbm.at[pl.ds((pbase + s) * 2, 2 * CHUNK)],
                        pts_v)
        pltpu.sync_copy(cost_hbm.at[pl.ds(pbase + s, CHUNK)], cost_v)

        def g_body(g, gcarry):
            pos = g * L + iota
            x = plsc.load_gather(pts_v, [pos * 2])
            y = plsc.load_gather(pts_v, [pos * 2 + 1])
            cst = plsc.load_gather(cost_v, [pos])
            ix = (x + 0.5).astype(jnp.int32)
            iy = (y + 0.5).astype(jnp.int32)
            lin = (iy << 10) + ix
            m = (ix < W) & (iy < H) & (pos >= thr) & ((lin >> 19) == cid)
            lidx = lin & (HALF - 1)
            cv = jnp.where(m, cst, 0.0)
            ov = jnp.where(m, 1.0, 0.0)
            js = jnp.zeros((L,), jnp.int32) + (g >> 3)
            p128 = pos & 127
            plsc.store_scatter(vals_s, [js, p128], cv)
            plsc.store_scatter(vals_c, [js, p128], ov)
            plsc.store_scatter(idx_v, [js, p128], lidx)
            return gcarry
        lax.fori_loop(0, GROUPS, g_body, 0)

        descs = []
        for j in range(NBATCH):
            descs.append(pltpu.async_copy(
                vals_s.at[j], acc_s.at[idx_v.at[j]], sem, add=True))
            descs.append(pltpu.async_copy(
                vals_c.at[j], acc_c.at[idx_v.at[j]], sem, add=True))
        for d in descs:
            d.wait()
        return carry
    lax.fori_loop(0, NCH, chunk_body, 0)

    plsc.subcore_barrier()

    gbase = cid * HALF + sid * RPT

    def f_body(fc, carry):
        lb = sid * RPT + fc * FCH
        pltpu.sync_copy(acc_s.at[pl.ds(lb, FCH)], fin_s)
        pltpu.sync_copy(acc_c.at[pl.ds(lb, FCH)], fin_c)

        def fg_body(g, gcarry):
            rows = g * L + iota
            sm = plsc.load_gather(fin_s, [rows])
            cnt = plsc.load_gather(fin_c, [rows])
            mean = sm / jnp.maximum(cnt, 1.0)
            outv = jnp.where(cnt > 0.0, mean, dflt)
            plsc.store_scatter(omap_v, [rows], outv)
            plsc.store_scatter(omask_v, [rows], cnt)
            return gcarry
        lax.fori_loop(0, FCH // L, fg_body, 0)

        gb = gbase + fc * FCH
        pltpu.sync_copy(omap_v, map_hbm.at[pl.ds(gb, FCH)])
        pltpu.sync_copy(omask_v, mask_hbm.at[pl.ds(gb, FCH)])
        return carry
    lax.fori_loop(0, NFCH, f_body, 0)


_mesh = plsc.VectorSubcoreMesh(core_axis_name="c", subcore_axis_name="s",
                               num_cores=NC, num_subcores=NS)

_cost_map_call = pl.kernel(
    _body,
    out_type=(jax.ShapeDtypeStruct((HW,), jnp.float32),
              jax.ShapeDtypeStruct((HW,), jnp.float32)),
    mesh=_mesh,
    compiler_params=pltpu.CompilerParams(needs_layout_passes=False),
    scratch_types=(
        pltpu.VMEM_SHARED((HALF,), jnp.float32),
        pltpu.VMEM_SHARED((HALF,), jnp.float32),
        pltpu.VMEM((2 * CHUNK,), jnp.float32),
        pltpu.VMEM((CHUNK,), jnp.float32),
        pltpu.VMEM((NBATCH, 128), jnp.float32),
        pltpu.VMEM((NBATCH, 128), jnp.float32),
        pltpu.VMEM((NBATCH, 128), jnp.int32),
        pltpu.VMEM((FCH,), jnp.float32),
        pltpu.VMEM((FCH,), jnp.float32),
        pltpu.VMEM((FCH,), jnp.float32),
        pltpu.VMEM((FCH,), jnp.float32),
        pltpu.VMEM((L,), jnp.float32),
        pltpu.VMEM((ZLEN,), jnp.float32),
        pltpu.SemaphoreType.DMA,
    ),
)


@jax.jit
def kernel(points, cost, default_cost):
    pts_flat = points.reshape(-1)
    dflt16 = jnp.broadcast_to(default_cost.astype(jnp.float32), (L,))
    cmap, cmask = _cost_map_call(pts_flat, cost, dflt16)
    return cmap.reshape(H, W), cmask.reshape(H, W)


# speedup vs baseline: 2.3477x; 1.0428x over previous
import jax
import jax.numpy as jnp
from jax import lax
from jax.experimental import pallas as pl
from jax.experimental.pallas import tpu as pltpu
from jax.experimental.pallas import tpu_sc as plsc

H = 1024
W = 1024
N = 2000000
HW = H * W
NC = 2
NS = 16
L = 16

Q = N // NS
CHUNK = 1024
GROUPS = CHUNK // L
NBATCH = CHUNK // 128
NCH = -(-Q // CHUNK)
LAST_S = Q - CHUNK

RPT = HW // NS
ZLEN = 4096


def _scatter_body(points_hbm, cost_hbm, grids_hbm,
                  acc, pts_v, cost_v, vals_v, idx_v, zsrc, sem):
    cid = lax.axis_index("c")
    sid = lax.axis_index("s")
    iota = lax.iota(jnp.int32, L)
    fzero = jnp.zeros((L,), jnp.float32)

    def z_body(g, carry):
        plsc.store_scatter(zsrc, [g * L + iota], fzero)
        return carry
    lax.fori_loop(0, ZLEN // L, z_body, 0)
    base0 = sid * RPT
    zdescs = []
    for k in range(RPT // ZLEN):
        zdescs.append(pltpu.async_copy(
            zsrc, acc.at[pl.ds(base0 + k * ZLEN, ZLEN)], sem))
    for d in zdescs:
        d.wait()

    plsc.subcore_barrier()

    pbase = sid * Q
    is_sum = (jnp.zeros((L,), jnp.int32) + cid) == 0

    def chunk_body(c, carry):
        s = jnp.minimum(c * CHUNK, LAST_S)
        thr = c * CHUNK - s
        dp = pltpu.async_copy(
            points_hbm.at[pl.ds((pbase + s) * 2, 2 * CHUNK)], pts_v, sem)
        dc = pltpu.async_copy(
            cost_hbm.at[pl.ds(pbase + s, CHUNK)], cost_v, sem)
        dp.wait()
        dc.wait()

        def g_body(g, gcarry):
            pos = g * L + iota
            x = plsc.load_gather(pts_v, [pos * 2])
            y = plsc.load_gather(pts_v, [pos * 2 + 1])
            cst = plsc.load_gather(cost_v, [pos])
            ix = (x + 0.5).astype(jnp.int32)
            iy = (y + 0.5).astype(jnp.int32)
            lin = (iy << 10) + ix
            m = (ix < W) & (iy < H) & (pos >= thr)
            lidx = lin & (HW - 1)
            val = jnp.where(is_sum, cst, 1.0)
            cv = jnp.where(m, val, 0.0)
            js = jnp.zeros((L,), jnp.int32) + (g >> 3)
            p128 = pos & 127
            plsc.store_scatter(vals_v, [js, p128], cv)
            plsc.store_scatter(idx_v, [js, p128], lidx)
            return gcarry
        lax.fori_loop(0, GROUPS, g_body, 0)

        descs = []
        for j in range(NBATCH):
            descs.append(pltpu.async_copy(
                vals_v.at[j], acc.at[idx_v.at[j]], sem, add=True))
        for d in descs:
            d.wait()
        return carry
    lax.fori_loop(0, NCH, chunk_body, 0)

    plsc.subcore_barrier()

    pltpu.sync_copy(acc.at[pl.ds(sid * RPT, RPT)],
                    grids_hbm.at[pl.ds(cid * HW + sid * RPT, RPT)])


_mesh = plsc.VectorSubcoreMesh(core_axis_name="c", subcore_axis_name="s",
                               num_cores=NC, num_subcores=NS)

_scatter_call = pl.kernel(
    _scatter_body,
    out_type=jax.ShapeDtypeStruct((NC * HW,), jnp.float32),
    mesh=_mesh,
    compiler_params=pltpu.CompilerParams(needs_layout_passes=False),
    scratch_types=(
        pltpu.VMEM_SHARED((HW,), jnp.float32),
        pltpu.VMEM((2 * CHUNK,), jnp.float32),
        pltpu.VMEM((CHUNK,), jnp.float32),
        pltpu.VMEM((NBATCH, 128), jnp.float32),
        pltpu.VMEM((NBATCH, 128), jnp.int32),
        pltpu.VMEM((ZLEN,), jnp.float32),
        pltpu.SemaphoreType.DMA,
    ),
)

BR = 128


def _finalize_body(sum_ref, cnt_ref, dflt_ref, map_ref, mask_ref):
    sm = sum_ref[...]
    cn = cnt_ref[...]
    mean = sm / jnp.maximum(cn, 1.0)
    map_ref[...] = jnp.where(cn > 0.0, mean, dflt_ref[0, 0])
    mask_ref[...] = cn


_finalize_call = pl.pallas_call(
    _finalize_body,
    grid=(H // BR,),
    in_specs=[pl.BlockSpec((BR, W), lambda i: (i, 0)),
              pl.BlockSpec((BR, W), lambda i: (i, 0)),
              pl.BlockSpec((1, 1), lambda i: (0, 0))],
    out_specs=[pl.BlockSpec((BR, W), lambda i: (i, 0)),
               pl.BlockSpec((BR, W), lambda i: (i, 0))],
    out_shape=(jax.ShapeDtypeStruct((H, W), jnp.float32),
               jax.ShapeDtypeStruct((H, W), jnp.float32)),
)


@jax.jit
def kernel(points, cost, default_cost):
    pts_flat = points.reshape(-1)
    grids = _scatter_call(pts_flat, cost).reshape(NC, H, W)
    cost_map, cost_mask = _finalize_call(
        grids[0], grids[1],
        default_cost.astype(jnp.float32).reshape(1, 1))
    return cost_map, cost_mask


# speedup vs baseline: 2.3971x; 1.0210x over previous
import jax
import jax.numpy as jnp
from jax import lax
from jax.experimental import pallas as pl
from jax.experimental.pallas import tpu as pltpu
from jax.experimental.pallas import tpu_sc as plsc

H = 1024
W = 1024
N = 2000000
HW = H * W
NC = 2
NS = 16
L = 16

Q = N // NS
CHUNK = 1024
GROUPS = CHUNK // L
NBATCH = CHUNK // 128
NCH = -(-Q // CHUNK)
LAST_S = Q - CHUNK

RPT = HW // NS
ZLEN = 4096


def _scatter_body(points_hbm, cost_hbm, grids_hbm,
                  acc, pts_v, cost_v, vals_v, idx_v, zsrc, sem):
    cid = lax.axis_index("c")
    sid = lax.axis_index("s")
    iota = lax.iota(jnp.int32, L)
    fzero = jnp.zeros((L,), jnp.float32)

    @plsc.parallel_loop(0, ZLEN // L, unroll=8)
    def z_body(g):
        zsrc[pl.ds(g * L, L)] = fzero
    base0 = sid * RPT
    zdescs = []
    for k in range(RPT // ZLEN):
        zdescs.append(pltpu.async_copy(
            zsrc, acc.at[pl.ds(base0 + k * ZLEN, ZLEN)], sem))
    for d in zdescs:
        d.wait()

    plsc.subcore_barrier()

    pbase = sid * Q
    is_sum = (jnp.zeros((L,), jnp.int32) + cid) == 0

    def chunk_body(c, carry):
        s = jnp.minimum(c * CHUNK, LAST_S)
        thr = c * CHUNK - s
        dp = pltpu.async_copy(
            points_hbm.at[pl.ds((pbase + s) * 2, 2 * CHUNK)], pts_v, sem)
        dc = pltpu.async_copy(
            cost_hbm.at[pl.ds(pbase + s, CHUNK)], cost_v, sem)
        dp.wait()
        dc.wait()

        @plsc.parallel_loop(0, GROUPS, unroll=8)
        def g_body(g):
            pos = g * L + iota
            x = plsc.load_gather(pts_v, [pos * 2])
            y = plsc.load_gather(pts_v, [pos * 2 + 1])
            cst = cost_v[pl.ds(g * L, L)]
            ix = (x + 0.5).astype(jnp.int32)
            iy = (y + 0.5).astype(jnp.int32)
            lin = (iy << 10) + ix
            m = (ix < W) & (iy < H) & (pos >= thr)
            lidx = lin & (HW - 1)
            val = jnp.where(is_sum, cst, 1.0)
            cv = jnp.where(m, val, 0.0)
            vals_v[g >> 3, pl.ds((g & 7) * L, L)] = cv
            idx_v[g >> 3, pl.ds((g & 7) * L, L)] = lidx

        descs = []
        for j in range(NBATCH):
            descs.append(pltpu.async_copy(
                vals_v.at[j], acc.at[idx_v.at[j]], sem, add=True))
        for d in descs:
            d.wait()
        return carry
    lax.fori_loop(0, NCH, chunk_body, 0)

    plsc.subcore_barrier()

    pltpu.sync_copy(acc.at[pl.ds(sid * RPT, RPT)],
                    grids_hbm.at[pl.ds(cid * HW + sid * RPT, RPT)])


_mesh = plsc.VectorSubcoreMesh(core_axis_name="c", subcore_axis_name="s",
                               num_cores=NC, num_subcores=NS)

_scatter_call = pl.kernel(
    _scatter_body,
    out_type=jax.ShapeDtypeStruct((NC * HW,), jnp.float32),
    mesh=_mesh,
    compiler_params=pltpu.CompilerParams(needs_layout_passes=False),
    scratch_types=(
        pltpu.VMEM_SHARED((HW,), jnp.float32),
        pltpu.VMEM((2 * CHUNK,), jnp.float32),
        pltpu.VMEM((CHUNK,), jnp.float32),
        pltpu.VMEM((NBATCH, 128), jnp.float32),
        pltpu.VMEM((NBATCH, 128), jnp.int32),
        pltpu.VMEM((ZLEN,), jnp.float32),
        pltpu.SemaphoreType.DMA,
    ),
)

BR = 128


def _finalize_body(sum_ref, cnt_ref, dflt_ref, map_ref, mask_ref):
    sm = sum_ref[...]
    cn = cnt_ref[...]
    mean = sm / jnp.maximum(cn, 1.0)
    map_ref[...] = jnp.where(cn > 0.0, mean, dflt_ref[0, 0])
    mask_ref[...] = cn


_finalize_call = pl.pallas_call(
    _finalize_body,
    grid=(H // BR,),
    in_specs=[pl.BlockSpec((BR, W), lambda i: (i, 0)),
              pl.BlockSpec((BR, W), lambda i: (i, 0)),
              pl.BlockSpec((1, 1), lambda i: (0, 0))],
    out_specs=[pl.BlockSpec((BR, W), lambda i: (i, 0)),
               pl.BlockSpec((BR, W), lambda i: (i, 0))],
    out_shape=(jax.ShapeDtypeStruct((H, W), jnp.float32),
               jax.ShapeDtypeStruct((H, W), jnp.float32)),
)


@jax.jit
def kernel(points, cost, default_cost):
    pts_flat = points.reshape(-1)
    grids = _scatter_call(pts_flat, cost).reshape(NC, H, W)
    cost_map, cost_mask = _finalize_call(
        grids[0], grids[1],
        default_cost.astype(jnp.float32).reshape(1, 1))
    return cost_map, cost_mask


# speedup vs baseline: 2.4782x; 1.0339x over previous
import jax
import jax.numpy as jnp
from jax import lax
from jax.experimental import pallas as pl
from jax.experimental.pallas import tpu as pltpu
from jax.experimental.pallas import tpu_sc as plsc

H = 1024
W = 1024
N = 2000000
HW = H * W
NC = 2
NS = 16
L = 16

Q = N // NS
CHUNK = 1024
GROUPS = CHUNK // L
NBATCH = CHUNK // 128
NCH = -(-Q // CHUNK)
LAST_S = Q - CHUNK

RPT = HW // NS
ZLEN = 4096


NCH2 = NCH + (NCH % 2)


def _scatter_body(points_hbm, cost_hbm, grids_hbm,
                  acc, pts_v0, pts_v1, cost_v0, cost_v1,
                  vals_v0, vals_v1, idx_v0, idx_v1, zsrc,
                  sem_z, sem_ld0, sem_ld1, sem_sc0, sem_sc1):
    pts_b = (pts_v0, pts_v1)
    cost_b = (cost_v0, cost_v1)
    vals_b = (vals_v0, vals_v1)
    idx_b = (idx_v0, idx_v1)
    cid = lax.axis_index("c")
    sid = lax.axis_index("s")
    iota = lax.iota(jnp.int32, L)
    fzero = jnp.zeros((L,), jnp.float32)

    @plsc.parallel_loop(0, ZLEN // L, unroll=8)
    def z_body(g):
        zsrc[pl.ds(g * L, L)] = fzero
    base0 = sid * RPT
    zdescs = []
    for k in range(RPT // ZLEN):
        zdescs.append(pltpu.async_copy(
            zsrc, acc.at[pl.ds(base0 + k * ZLEN, ZLEN)], sem_z))

    for vv, iv in ((vals_v0, idx_v0), (vals_v1, idx_v1)):
        @plsc.parallel_loop(0, NBATCH * 8, unroll=8)
        def s_body(t, vv=vv, iv=iv):
            vv[t >> 3, pl.ds((t & 7) * L, L)] = fzero
            iv[t >> 3, pl.ds((t & 7) * L, L)] = t * L + iota

    for d in zdescs:
        d.wait()
    plsc.subcore_barrier()

    pbase = sid * Q
    is_sum = (jnp.zeros((L,), jnp.int32) + cid) == 0
    ld_sems = (sem_ld0, sem_ld1)
    sc_sems = (sem_sc0, sem_sc1)

    def issue_loads(c, b, sem):
        s = jnp.minimum(c * CHUNK, LAST_S)
        pltpu.async_copy(points_hbm.at[pl.ds((pbase + s) * 2, 2 * CHUNK)],
                         pts_b[b], sem)
        pltpu.async_copy(cost_hbm.at[pl.ds(pbase + s, CHUNK)],
                         cost_b[b], sem)

    def wait_loads(b, sem):
        pltpu.make_async_copy(points_hbm.at[pl.ds(0, 2 * CHUNK)],
                              pts_b[b], sem).wait()
        pltpu.make_async_copy(cost_hbm.at[pl.ds(0, CHUNK)],
                              cost_b[b], sem).wait()

    def issue_scatters(b, sem):
        for j in range(NBATCH):
            pltpu.async_copy(vals_b[b].at[j],
                             acc.at[idx_b[b].at[j]], sem, add=True)

    def drain_scatters(b, sem):
        for j in range(NBATCH):
            pltpu.make_async_copy(vals_b[b].at[j],
                                  acc.at[idx_b[b].at[j]], sem).wait()

    issue_loads(0, 0, sem_ld0)
    issue_loads(1, 1, sem_ld1)
    issue_scatters(0, sem_sc0)
    issue_scatters(1, sem_sc1)

    def step(c, b):
        wait_loads(b, ld_sems[b])
        drain_scatters(b, sc_sems[b])
        s = jnp.minimum(c * CHUNK, LAST_S)
        thr = c * CHUNK - s

        @plsc.parallel_loop(0, GROUPS, unroll=8)
        def g_body(g):
            pos = g * L + iota
            x = plsc.load_gather(pts_b[b], [pos * 2])
            y = plsc.load_gather(pts_b[b], [pos * 2 + 1])
            cst = cost_b[b][pl.ds(g * L, L)]
            ix = (x + 0.5).astype(jnp.int32)
            iy = (y + 0.5).astype(jnp.int32)
            lin = (iy << 10) + ix
            m = (ix < W) & (iy < H) & (pos >= thr)
            lidx = lin & (HW - 1)
            val = jnp.where(is_sum, cst, 1.0)
            cv = jnp.where(m, val, 0.0)
            vals_b[b][g >> 3, pl.ds((g & 7) * L, L)] = cv
            idx_b[b][g >> 3, pl.ds((g & 7) * L, L)] = lidx

        issue_scatters(b, sc_sems[b])
        issue_loads(c + 2, b, ld_sems[b])

    def chunk_body(k2, carry):
        step(2 * k2, 0)
        step(2 * k2 + 1, 1)
        return carry
    lax.fori_loop(0, NCH2 // 2, chunk_body, 0)

    drain_scatters(0, sem_sc0)
    drain_scatters(1, sem_sc1)
    wait_loads(0, sem_ld0)
    wait_loads(1, sem_ld1)

    plsc.subcore_barrier()

    pltpu.sync_copy(acc.at[pl.ds(sid * RPT, RPT)],
                    grids_hbm.at[pl.ds(cid * HW + sid * RPT, RPT)])


_mesh = plsc.VectorSubcoreMesh(core_axis_name="c", subcore_axis_name="s",
                               num_cores=NC, num_subcores=NS)

_scatter_call = pl.kernel(
    _scatter_body,
    out_type=jax.ShapeDtypeStruct((NC * HW,), jnp.float32),
    mesh=_mesh,
    compiler_params=pltpu.CompilerParams(needs_layout_passes=False),
    scratch_types=(
        pltpu.VMEM_SHARED((HW,), jnp.float32),
        pltpu.VMEM((2 * CHUNK,), jnp.float32),
        pltpu.VMEM((2 * CHUNK,), jnp.float32),
        pltpu.VMEM((CHUNK,), jnp.float32),
        pltpu.VMEM((CHUNK,), jnp.float32),
        pltpu.VMEM((NBATCH, 128), jnp.float32),
        pltpu.VMEM((NBATCH, 128), jnp.float32),
        pltpu.VMEM((NBATCH, 128), jnp.int32),
        pltpu.VMEM((NBATCH, 128), jnp.int32),
        pltpu.VMEM((ZLEN,), jnp.float32),
        pltpu.SemaphoreType.DMA,
        pltpu.SemaphoreType.DMA,
        pltpu.SemaphoreType.DMA,
        pltpu.SemaphoreType.DMA,
        pltpu.SemaphoreType.DMA,
    ),
)

BR = 128


def _finalize_body(sum_ref, cnt_ref, dflt_ref, map_ref, mask_ref):
    sm = sum_ref[...]
    cn = cnt_ref[...]
    mean = sm / jnp.maximum(cn, 1.0)
    map_ref[...] = jnp.where(cn > 0.0, mean, dflt_ref[0, 0])
    mask_ref[...] = cn


_finalize_call = pl.pallas_call(
    _finalize_body,
    grid=(H // BR,),
    in_specs=[pl.BlockSpec((BR, W), lambda i: (i, 0)),
              pl.BlockSpec((BR, W), lambda i: (i, 0)),
              pl.BlockSpec((1, 1), lambda i: (0, 0))],
    out_specs=[pl.BlockSpec((BR, W), lambda i: (i, 0)),
               pl.BlockSpec((BR, W), lambda i: (i, 0))],
    out_shape=(jax.ShapeDtypeStruct((H, W), jnp.float32),
               jax.ShapeDtypeStruct((H, W), jnp.float32)),
)


@jax.jit
def kernel(points, cost, default_cost):
    pts_flat = points.reshape(-1)
    grids = _scatter_call(pts_flat, cost).reshape(NC, H, W)
    cost_map, cost_mask = _finalize_call(
        grids[0], grids[1],
        default_cost.astype(jnp.float32).reshape(1, 1))
    return cost_map, cost_mask


# speedup vs baseline: 2.4824x; 1.0017x over previous
import jax
import jax.numpy as jnp
from jax import lax
from jax.experimental import pallas as pl
from jax.experimental.pallas import tpu as pltpu
from jax.experimental.pallas import tpu_sc as plsc

H = 1024
W = 1024
N = 2000000
HW = H * W
NC = 2
NS = 16
L = 16

Q = N // NS
CHUNK = 1024
GROUPS = CHUNK // L
NBATCH = CHUNK // 128
NCH = -(-Q // CHUNK)
LAST_S = Q - CHUNK

RPT = HW // NS
ZLEN = 4096


NCH2 = NCH + (NCH % 2)


def _scatter_body(points_hbm, cost_hbm, grids_hbm,
                  acc, pts_v0, pts_v1, cost_v0, cost_v1,
                  vals_v0, vals_v1, idx_v0, idx_v1, zsrc,
                  sem_z, sem_ld0, sem_ld1, sem_sc0, sem_sc1):
    pts_b = (pts_v0, pts_v1)
    cost_b = (cost_v0, cost_v1)
    vals_b = (vals_v0, vals_v1)
    idx_b = (idx_v0, idx_v1)
    cid = lax.axis_index("c")
    sid = lax.axis_index("s")
    iota = lax.iota(jnp.int32, L)
    fzero = jnp.zeros((L,), jnp.float32)

    @plsc.parallel_loop(0, ZLEN // L, unroll=8)
    def z_body(g):
        zsrc[pl.ds(g * L, L)] = fzero
    base0 = sid * RPT
    zdescs = []
    for k in range(RPT // ZLEN):
        zdescs.append(pltpu.async_copy(
            zsrc, acc.at[pl.ds(base0 + k * ZLEN, ZLEN)], sem_z))

    for vv, iv in ((vals_v0, idx_v0), (vals_v1, idx_v1)):
        @plsc.parallel_loop(0, NBATCH * 8, unroll=8)
        def s_body(t, vv=vv, iv=iv):
            vv[t >> 3, pl.ds((t & 7) * L, L)] = fzero
            iv[t >> 3, pl.ds((t & 7) * L, L)] = t * L + iota

    for d in zdescs:
        d.wait()
    plsc.subcore_barrier()

    pbase = sid * Q
    is_sum = (jnp.zeros((L,), jnp.int32) + cid) == 0
    ld_sems = (sem_ld0, sem_ld1)
    sc_sems = (sem_sc0, sem_sc1)

    def issue_loads(c, b, sem):
        s = jnp.minimum(c * CHUNK, LAST_S)
        pltpu.async_copy(points_hbm.at[pl.ds((pbase + s) * 2, 2 * CHUNK)],
                         pts_b[b], sem)
        pltpu.async_copy(cost_hbm.at[pl.ds(pbase + s, CHUNK)],
                         cost_b[b], sem)

    def wait_loads(b, sem):
        pltpu.make_async_copy(points_hbm.at[pl.ds(0, 2 * CHUNK)],
                              pts_b[b], sem).wait()
        pltpu.make_async_copy(cost_hbm.at[pl.ds(0, CHUNK)],
                              cost_b[b], sem).wait()

    def issue_scatters(b, sem):
        for j in range(NBATCH):
            pltpu.async_copy(vals_b[b].at[j],
                             acc.at[idx_b[b].at[j]], sem, add=True)

    def drain_scatters(b, sem):
        for j in range(NBATCH):
            pltpu.make_async_copy(vals_b[b].at[j],
                                  acc.at[idx_b[b].at[j]], sem).wait()

    issue_loads(0, 0, sem_ld0)
    issue_loads(1, 1, sem_ld1)
    issue_scatters(0, sem_sc0)
    issue_scatters(1, sem_sc1)

    def step(c, b):
        wait_loads(b, ld_sems[b])
        drain_scatters(b, sc_sems[b])
        s = jnp.minimum(c * CHUNK, LAST_S)
        thr = c * CHUNK - s

        @plsc.parallel_loop(0, GROUPS, unroll=8)
        def g_body(g):
            pos = g * L + iota
            x = plsc.load_gather(pts_b[b], [pos * 2])
            y = plsc.load_gather(pts_b[b], [pos * 2 + 1])
            cst = cost_b[b][pl.ds(g * L, L)]
            ix = (x + 0.5).astype(jnp.int32)
            iy = (y + 0.5).astype(jnp.int32)
            lin = (iy << 10) + ix
            m = (ix < W) & (iy < H) & (pos >= thr)
            lidx = lin & (HW - 1)
            val = jnp.where(is_sum, cst, 1.0)
            cv = jnp.where(m, val, 0.0)
            vals_b[b][g >> 3, pl.ds((g & 7) * L, L)] = cv
            idx_b[b][g >> 3, pl.ds((g & 7) * L, L)] = lidx

        issue_scatters(b, sc_sems[b])
        issue_loads(c + 2, b, ld_sems[b])

    def chunk_body(k2, carry):
        step(2 * k2, 0)
        step(2 * k2 + 1, 1)
        return carry
    lax.fori_loop(0, NCH2 // 2, chunk_body, 0)

    drain_scatters(0, sem_sc0)
    drain_scatters(1, sem_sc1)
    wait_loads(0, sem_ld0)
    wait_loads(1, sem_ld1)

    plsc.subcore_barrier()

    pltpu.sync_copy(acc.at[pl.ds(sid * RPT, RPT)],
                    grids_hbm.at[pl.ds(cid * HW + sid * RPT, RPT)])


_mesh = plsc.VectorSubcoreMesh(core_axis_name="c", subcore_axis_name="s",
                               num_cores=NC, num_subcores=NS)

_scatter_call = pl.kernel(
    _scatter_body,
    out_type=jax.ShapeDtypeStruct((NC * HW,), jnp.float32),
    mesh=_mesh,
    compiler_params=pltpu.CompilerParams(needs_layout_passes=False,
                                         use_tc_tiling_on_sc=True),
    scratch_types=(
        pltpu.VMEM_SHARED((HW,), jnp.float32),
        pltpu.VMEM((2 * CHUNK,), jnp.float32),
        pltpu.VMEM((2 * CHUNK,), jnp.float32),
        pltpu.VMEM((CHUNK,), jnp.float32),
        pltpu.VMEM((CHUNK,), jnp.float32),
        pltpu.VMEM((NBATCH, 128), jnp.float32),
        pltpu.VMEM((NBATCH, 128), jnp.float32),
        pltpu.VMEM((NBATCH, 128), jnp.int32),
        pltpu.VMEM((NBATCH, 128), jnp.int32),
        pltpu.VMEM((ZLEN,), jnp.float32),
        pltpu.SemaphoreType.DMA,
        pltpu.SemaphoreType.DMA,
        pltpu.SemaphoreType.DMA,
        pltpu.SemaphoreType.DMA,
        pltpu.SemaphoreType.DMA,
    ),
)

BR = 128


def _finalize_body(sum_ref, cnt_ref, dflt_ref, map_ref, mask_ref):
    sm = sum_ref[...]
    cn = cnt_ref[...]
    mean = sm / jnp.maximum(cn, 1.0)
    map_ref[...] = jnp.where(cn > 0.0, mean, dflt_ref[0, 0])
    mask_ref[...] = cn


_finalize_call = pl.pallas_call(
    _finalize_body,
    grid=(H // BR,),
    in_specs=[pl.BlockSpec((BR, W), lambda i: (i, 0)),
              pl.BlockSpec((BR, W), lambda i: (i, 0)),
              pl.BlockSpec((1, 1), lambda i: (0, 0))],
    out_specs=[pl.BlockSpec((BR, W), lambda i: (i, 0)),
               pl.BlockSpec((BR, W), lambda i: (i, 0))],
    out_shape=(jax.ShapeDtypeStruct((H, W), jnp.float32),
               jax.ShapeDtypeStruct((H, W), jnp.float32)),
)


@jax.jit
def kernel(points, cost, default_cost):
    pts_flat = points.reshape(-1)
    grids = _scatter_call(pts_flat, cost).reshape(NC, H, W)
    cost_map, cost_mask = _finalize_call(
        grids[0], grids[1],
        default_cost.astype(jnp.float32).reshape(1, 1))
    return cost_map, cost_mask


# speedup vs baseline: 44.8080x; 18.0505x over previous
import jax
import jax.numpy as jnp
from jax import lax
from jax.experimental import pallas as pl
from jax.experimental.pallas import tpu as pltpu
from jax.experimental.pallas import tpu_sc as plsc

H = 1024
W = 1024
N = 2000000
HW = H * W
NC = 2
NS = 16
L = 16

Q = N // NS
CHUNK = 1024
GROUPS = CHUNK // L
NBATCH = CHUNK // 128
NCH = -(-Q // CHUNK)
LAST_S = Q - CHUNK

RPT = HW // NS
ZLEN = 4096


NCH2 = NCH + (NCH % 2)


def _scatter_body(points_hbm, cost_hbm, grids_hbm,
                  acc, pts_v0, pts_v1, cost_v0, cost_v1,
                  vals_v0, vals_v1, idx_v0, idx_v1, zsrc,
                  sem_z, sem_ld0, sem_ld1, sem_sc0, sem_sc1):
    pts_b = (pts_v0, pts_v1)
    cost_b = (cost_v0, cost_v1)
    vals_b = (vals_v0, vals_v1)
    idx_b = (idx_v0, idx_v1)
    cid = lax.axis_index("c")
    sid = lax.axis_index("s")
    iota = lax.iota(jnp.int32, L)
    fzero = jnp.zeros((L,), jnp.float32)

    @plsc.parallel_loop(0, ZLEN // L, unroll=8)
    def z_body(g):
        zsrc[pl.ds(g * L, L)] = fzero
    base0 = sid * RPT
    zdescs = []
    for k in range(RPT // ZLEN):
        zdescs.append(pltpu.async_copy(
            zsrc, acc.at[pl.ds(base0 + k * ZLEN, ZLEN)], sem_z))

    for vv, iv in ((vals_v0, idx_v0), (vals_v1, idx_v1)):
        @plsc.parallel_loop(0, NBATCH * 8, unroll=8)
        def s_body(t, vv=vv, iv=iv):
            vv[t >> 3, pl.ds((t & 7) * L, L)] = fzero
            iv[t >> 3, pl.ds((t & 7) * L, L)] = t * L + iota

    for d in zdescs:
        d.wait()
    plsc.subcore_barrier()

    pbase = sid * Q
    is_sum = (jnp.zeros((L,), jnp.int32) + cid) == 0
    ld_sems = (sem_ld0, sem_ld1)
    sc_sems = (sem_sc0, sem_sc1)

    def issue_loads(c, b, sem):
        s = jnp.minimum(c * CHUNK, LAST_S)
        pltpu.async_copy(points_hbm.at[pl.ds(pbase + s, CHUNK)],
                         pts_b[b].at[pl.ds(0, CHUNK)], sem)
        pltpu.async_copy(points_hbm.at[pl.ds(N + pbase + s, CHUNK)],
                         pts_b[b].at[pl.ds(CHUNK, CHUNK)], sem)
        pltpu.async_copy(cost_hbm.at[pl.ds(pbase + s, CHUNK)],
                         cost_b[b], sem)

    def wait_loads(b, sem):
        pltpu.make_async_copy(points_hbm.at[pl.ds(0, CHUNK)],
                              pts_b[b].at[pl.ds(0, CHUNK)], sem).wait()
        pltpu.make_async_copy(points_hbm.at[pl.ds(0, CHUNK)],
                              pts_b[b].at[pl.ds(CHUNK, CHUNK)], sem).wait()
        pltpu.make_async_copy(cost_hbm.at[pl.ds(0, CHUNK)],
                              cost_b[b], sem).wait()

    def issue_scatters(b, sem):
        for j in range(NBATCH):
            pltpu.async_copy(vals_b[b].at[j],
                             acc.at[idx_b[b].at[j]], sem, add=True)

    def drain_scatters(b, sem):
        for j in range(NBATCH):
            pltpu.make_async_copy(vals_b[b].at[j],
                                  acc.at[idx_b[b].at[j]], sem).wait()

    issue_loads(0, 0, sem_ld0)
    issue_loads(1, 1, sem_ld1)
    issue_scatters(0, sem_sc0)
    issue_scatters(1, sem_sc1)

    def step(c, b):
        wait_loads(b, ld_sems[b])
        drain_scatters(b, sc_sems[b])
        s = jnp.minimum(c * CHUNK, LAST_S)
        thr = c * CHUNK - s

        @plsc.parallel_loop(0, GROUPS, unroll=8)
        def g_body(g):
            pos = g * L + iota
            x = pts_b[b][pl.ds(g * L, L)]
            y = pts_b[b][pl.ds(CHUNK + g * L, L)]
            cst = cost_b[b][pl.ds(g * L, L)]
            ix = (x + 0.5).astype(jnp.int32)
            iy = (y + 0.5).astype(jnp.int32)
            lin = (iy << 10) + ix
            m = (ix < W) & (iy < H) & (pos >= thr)
            lidx = lin & (HW - 1)
            val = jnp.where(is_sum, cst, 1.0)
            cv = jnp.where(m, val, 0.0)
            vals_b[b][g >> 3, pl.ds((g & 7) * L, L)] = cv
            idx_b[b][g >> 3, pl.ds((g & 7) * L, L)] = lidx

        issue_scatters(b, sc_sems[b])
        issue_loads(c + 2, b, ld_sems[b])

    def chunk_body(k2, carry):
        step(2 * k2, 0)
        step(2 * k2 + 1, 1)
        return carry
    lax.fori_loop(0, NCH2 // 2, chunk_body, 0)

    drain_scatters(0, sem_sc0)
    drain_scatters(1, sem_sc1)
    wait_loads(0, sem_ld0)
    wait_loads(1, sem_ld1)

    plsc.subcore_barrier()

    pltpu.sync_copy(acc.at[pl.ds(sid * RPT, RPT)],
                    grids_hbm.at[pl.ds(cid * HW + sid * RPT, RPT)])


_mesh = plsc.VectorSubcoreMesh(core_axis_name="c", subcore_axis_name="s",
                               num_cores=NC, num_subcores=NS)

_scatter_call = pl.kernel(
    _scatter_body,
    out_type=jax.ShapeDtypeStruct((NC * HW,), jnp.float32),
    mesh=_mesh,
    compiler_params=pltpu.CompilerParams(needs_layout_passes=False,
                                         use_tc_tiling_on_sc=True),
    scratch_types=(
        pltpu.VMEM_SHARED((HW,), jnp.float32),
        pltpu.VMEM((2 * CHUNK,), jnp.float32),
        pltpu.VMEM((2 * CHUNK,), jnp.float32),
        pltpu.VMEM((CHUNK,), jnp.float32),
        pltpu.VMEM((CHUNK,), jnp.float32),
        pltpu.VMEM((NBATCH, 128), jnp.float32),
        pltpu.VMEM((NBATCH, 128), jnp.float32),
        pltpu.VMEM((NBATCH, 128), jnp.int32),
        pltpu.VMEM((NBATCH, 128), jnp.int32),
        pltpu.VMEM((ZLEN,), jnp.float32),
        pltpu.SemaphoreType.DMA,
        pltpu.SemaphoreType.DMA,
        pltpu.SemaphoreType.DMA,
        pltpu.SemaphoreType.DMA,
        pltpu.SemaphoreType.DMA,
    ),
)

BR = 128


def _finalize_body(sum_ref, cnt_ref, dflt_ref, map_ref, mask_ref):
    sm = sum_ref[...]
    cn = cnt_ref[...]
    mean = sm / jnp.maximum(cn, 1.0)
    map_ref[...] = jnp.where(cn > 0.0, mean, dflt_ref[0, 0])
    mask_ref[...] = cn


_finalize_call = pl.pallas_call(
    _finalize_body,
    grid=(H // BR,),
    in_specs=[pl.BlockSpec((BR, W), lambda i: (i, 0)),
              pl.BlockSpec((BR, W), lambda i: (i, 0)),
              pl.BlockSpec((1, 1), lambda i: (0, 0))],
    out_specs=[pl.BlockSpec((BR, W), lambda i: (i, 0)),
               pl.BlockSpec((BR, W), lambda i: (i, 0))],
    out_shape=(jax.ShapeDtypeStruct((H, W), jnp.float32),
               jax.ShapeDtypeStruct((H, W), jnp.float32)),
)


@jax.jit
def kernel(points, cost, default_cost):
    pts_flat = points.T.reshape(-1)
    grids = _scatter_call(pts_flat, cost).reshape(NC, H, W)
    cost_map, cost_mask = _finalize_call(
        grids[0], grids[1],
        default_cost.astype(jnp.float32).reshape(1, 1))
    return cost_map, cost_mask
